# Initial kernel scaffold; baseline (speedup 1.0000x reference)
import jax
import jax.numpy as jnp
from jax.experimental import pallas as pl


def kernel(features, edge_index, edge_weight, W1, W2, W3, Tw, Tb, T1w, T1b, T2w, T2b):
    raise NotImplementedError("write your pallas kernel here")


# speedup vs baseline: 3.9041x; 3.9041x over previous
import functools

import jax
import jax.numpy as jnp
from jax import lax
from jax.experimental import pallas as pl
from jax.experimental.pallas import tpu as pltpu
from jax.experimental.pallas import tpu_sc as plsc

_NC = 2
_NS = 16
_NW = _NC * _NS
_L = 16


@functools.cache
def _make_spmm(n, e, d, k):
    e_per_tile = e // _NW
    n_chunks = e_per_tile // k
    assert n_chunks * k == e_per_tile
    n_row_chunks = n // k
    assert n_row_chunks * k == n and k % 8 == 0
    row_iters = -(-n_row_chunks // _NS)

    mesh = plsc.VectorSubcoreMesh(core_axis_name="c", subcore_axis_name="s")

    @functools.partial(
        pl.kernel,
        out_type=jax.ShapeDtypeStruct((_NC, n, d), jnp.float32),
        mesh=mesh,
        scratch_types=[
            pltpu.VMEM((n_chunks, k), jnp.int32),
            pltpu.VMEM((n_chunks, k), jnp.int32),
            pltpu.VMEM((n_chunks, k), jnp.float32),
            pltpu.VMEM((k, d), jnp.float32),
            pltpu.VMEM_SHARED((n, d), jnp.float32),
            pltpu.SemaphoreType.DMA,
        ],
        compiler_params=pltpu.CompilerParams(use_tc_tiling_on_sc=False),
    )
    def spmm(x_hbm, src_hbm, dst_hbm, w_hbm, out_hbm,
             src_v, dst_v, w_v, rows_v, acc_sh, sem):
        c = lax.axis_index("c")
        s = lax.axis_index("s")
        wid = c * _NS + s

        pltpu.sync_copy(src_hbm.at[wid], src_v)
        pltpu.sync_copy(dst_hbm.at[wid], dst_v)
        pltpu.sync_copy(w_hbm.at[wid], w_v)

        def zrow(i, carry):
            for j in range(d // _L):
                rows_v[i, pl.ds(j * _L, _L)] = jnp.zeros((_L,), jnp.float32)
            return carry
        lax.fori_loop(0, k, zrow, 0)

        def zcp(it, carry):
            idx = it * _NS + s
            @pl.when(idx < n_row_chunks)
            def _():
                r0 = pl.multiple_of(idx * k, 8)
                pltpu.sync_copy(rows_v, acc_sh.at[pl.ds(r0, k)])
            return carry
        lax.fori_loop(0, row_iters, zcp, 0)
        plsc.subcore_barrier()

        def chunk(ci, carry):
            pltpu.async_copy(x_hbm.at[src_v.at[ci]], rows_v, sem).wait()

            def scale(gi, carry2):
                wvec = w_v[ci, pl.ds(gi * _L, _L)]
                for ii in range(_L):
                    w = wvec[ii]
                    row = gi * _L + ii
                    for j in range(d // _L):
                        sl = pl.ds(j * _L, _L)
                        rows_v[row, sl] = rows_v[row, sl] * w
                return carry2
            lax.fori_loop(0, k // _L, scale, 0)

            pltpu.sync_copy(rows_v, acc_sh.at[dst_v.at[ci]], add=True)
            return carry
        lax.fori_loop(0, n_chunks, chunk, 0)
        plsc.subcore_barrier()

        def ocp(it, carry):
            idx = it * _NS + s
            @pl.when(idx < n_row_chunks)
            def _():
                r0 = pl.multiple_of(idx * k, 8)
                pltpu.sync_copy(acc_sh.at[pl.ds(r0, k)],
                                out_hbm.at[c, pl.ds(r0, k)])
            return carry
        lax.fori_loop(0, row_iters, ocp, 0)

    return spmm


def _spmm(x, src3, dst3, w3, n, e, d, k):
    return _make_spmm(n, e, d, k)(x, src3, dst3, w3)


_BR = 1000


def _proj(x, w):
    n, f = x.shape
    dh = w.shape[1]

    def body(x_ref, w_ref, o_ref):
        o_ref[...] = jnp.dot(x_ref[...], w_ref[...],
                             preferred_element_type=jnp.float32)

    return pl.pallas_call(
        body,
        grid=(n // _BR,),
        in_specs=[
            pl.BlockSpec((_BR, f), lambda i: (i, 0)),
            pl.BlockSpec((f, dh), lambda i: (0, 0)),
        ],
        out_specs=pl.BlockSpec((_BR, dh), lambda i: (i, 0)),
        out_shape=jax.ShapeDtypeStruct((n, dh), jnp.float32),
    )(x, w)


def _fused_trans(g, feat, gt, ft, b, w):
    _, n, dg = g.shape
    f = feat.shape[1]
    dh = gt.shape[1]
    dn = dh if w is None else w.shape[1]

    def body(*refs):
        if w is None:
            g_ref, f_ref, gt_ref, ft_ref, b_ref, o_ref = refs
        else:
            g_ref, f_ref, gt_ref, ft_ref, b_ref, w_ref, o_ref = refs
        gs = g_ref[0] + g_ref[1]
        h = jnp.dot(gs, gt_ref[...], preferred_element_type=jnp.float32)
        h += jnp.dot(f_ref[...], ft_ref[...], preferred_element_type=jnp.float32)
        h = jnp.maximum(h + b_ref[...], 0.0)
        if w is None:
            o_ref[...] = h
        else:
            o_ref[...] = jnp.dot(h, w_ref[...], preferred_element_type=jnp.float32)

    in_specs = [
        pl.BlockSpec((2, _BR, dg), lambda i: (0, i, 0)),
        pl.BlockSpec((_BR, f), lambda i: (i, 0)),
        pl.BlockSpec((dg, dh), lambda i: (0, 0)),
        pl.BlockSpec((f, dh), lambda i: (0, 0)),
        pl.BlockSpec((1, dh), lambda i: (0, 0)),
    ]
    args = [g, feat, gt, ft, b]
    if w is not None:
        in_specs.append(pl.BlockSpec((dh, dn), lambda i: (0, 0)))
        args.append(w)

    return pl.pallas_call(
        body,
        grid=(n // _BR,),
        in_specs=in_specs,
        out_specs=pl.BlockSpec((_BR, dn), lambda i: (i, 0)),
        out_shape=jax.ShapeDtypeStruct((n, dn), jnp.float32),
    )(*args)


def kernel(features, edge_index, edge_weight, W1, W2, W3, Tw, Tb, T1w, T1b, T2w, T2b):
    n, f = features.shape
    e = edge_weight.shape[0]
    h1 = W1.shape[1]
    h2 = W2.shape[1]
    h3 = W3.shape[1]
    k = 80
    n_chunks = e // _NW // k

    src3 = edge_index[0].reshape(_NW, n_chunks, k)
    dst3 = edge_index[1].reshape(_NW, n_chunks, k)
    w3 = edge_weight.reshape(_NW, n_chunks, k)

    TwgT, TwfT = Tw[:, :h1].T, Tw[:, h1:].T
    T1wgT, T1wfT = T1w[:, :h2].T, T1w[:, h2:].T
    T2wgT, T2wfT = T2w[:, :h3].T, T2w[:, h3:].T

    x1 = _proj(features, W1)
    g1a = _spmm(x1[:, :h1 // 2], src3, dst3, w3, n, e, h1 // 2, k)
    g1b = _spmm(x1[:, h1 // 2:], src3, dst3, w3, n, e, h1 // 2, k)
    g1 = jnp.concatenate([g1a, g1b], axis=2)
    x2 = _fused_trans(g1, features, TwgT, TwfT, Tb.reshape(1, h1), W2)
    g2 = _spmm(x2, src3, dst3, w3, n, e, h2, k)
    x3 = _fused_trans(g2, features, T1wgT, T1wfT, T1b.reshape(1, h2), W3)
    g3 = _spmm(x3, src3, dst3, w3, n, e, h3, k)
    return _fused_trans(g3, features, T2wgT, T2wfT, T2b.reshape(1, h3), None)


# speedup vs baseline: 6.2860x; 1.6101x over previous
import functools

import jax
import jax.numpy as jnp
from jax import lax
from jax.experimental import pallas as pl
from jax.experimental.pallas import tpu as pltpu
from jax.experimental.pallas import tpu_sc as plsc

_NC = 2
_NS = 16
_NW = _NC * _NS
_L = 16


@functools.cache
def _make_spmm(n, e, d, k):
    e_per_tile = e // _NW
    n_chunks = e_per_tile // k
    assert n_chunks * k == e_per_tile
    n_row_chunks = n // k
    assert n_row_chunks * k == n and k % 8 == 0
    row_iters = -(-n_row_chunks // _NS)

    mesh = plsc.VectorSubcoreMesh(core_axis_name="c", subcore_axis_name="s")

    @functools.partial(
        pl.kernel,
        out_type=jax.ShapeDtypeStruct((_NC, n, d), jnp.float32),
        mesh=mesh,
        scratch_types=[
            pltpu.VMEM((n_chunks, k), jnp.int32),
            pltpu.VMEM((n_chunks, k), jnp.int32),
            pltpu.VMEM((n_chunks, k), jnp.float32),
            pltpu.VMEM((k, d), jnp.float32),
            pltpu.VMEM((k, d), jnp.float32),
            pltpu.VMEM((k, d), jnp.float32),
            pltpu.SemaphoreType.DMA,
            pltpu.SemaphoreType.DMA,
            pltpu.SemaphoreType.DMA,
            pltpu.SemaphoreType.DMA,
            pltpu.SemaphoreType.DMA,
            pltpu.SemaphoreType.DMA,
            pltpu.VMEM_SHARED((n, d), jnp.float32),
        ],
        compiler_params=pltpu.CompilerParams(use_tc_tiling_on_sc=False),
    )
    def spmm(x_hbm, src_hbm, dst_hbm, w_hbm, out_hbm,
             src_v, dst_v, w_v, rows0, rows1, rows2,
             gs0, gs1, gs2, ss0, ss1, ss2, acc_sh):
        rows = [rows0, rows1, rows2]
        gsem = [gs0, gs1, gs2]
        ssem = [ss0, ss1, ss2]
        rows_v = rows0
        c = lax.axis_index("c")
        s = lax.axis_index("s")
        wid = c * _NS + s

        pltpu.sync_copy(src_hbm.at[wid], src_v)
        pltpu.sync_copy(dst_hbm.at[wid], dst_v)
        pltpu.sync_copy(w_hbm.at[wid], w_v)

        def zrow(i, carry):
            for j in range(d // _L):
                rows_v[i, pl.ds(j * _L, _L)] = jnp.zeros((_L,), jnp.float32)
            return carry
        lax.fori_loop(0, k, zrow, 0)

        def zcp(it, carry):
            idx = it * _NS + s
            @pl.when(idx < n_row_chunks)
            def _():
                r0 = pl.multiple_of(idx * k, 8)
                pltpu.sync_copy(rows_v, acc_sh.at[pl.ds(r0, k)])
            return carry
        lax.fori_loop(0, row_iters, zcp, 0)
        plsc.subcore_barrier()

        def scale(b, ci):
            def body(gi, carry2):
                wvec = w_v[ci, pl.ds(gi * _L, _L)]
                for ii in range(_L):
                    w = wvec[ii]
                    row = gi * _L + ii
                    for j in range(d // _L):
                        sl = pl.ds(j * _L, _L)
                        rows[b][row, sl] = rows[b][row, sl] * w
                return carry2
            lax.fori_loop(0, k // _L, body, 0)

        def gissue(b, ci):
            pltpu.async_copy(x_hbm.at[src_v.at[ci]], rows[b], gsem[b])

        def gwait(b, ci):
            pltpu.make_async_copy(x_hbm.at[src_v.at[ci]], rows[b], gsem[b]).wait()

        def sissue(b, ci):
            pltpu.async_copy(rows[b], acc_sh.at[dst_v.at[ci]], ssem[b], add=True)

        def swait(b, ci):
            pltpu.make_async_copy(rows[b], acc_sh.at[dst_v.at[ci]], ssem[b]).wait()

        gissue(0, 0)
        gissue(1, 1)

        n_iters = -(-(n_chunks + 1) // 3)

        def pipe(g, carry):
            for b in range(3):
                ci = g * 3 + b
                p = (b + 2) % 3

                @pl.when(ci < n_chunks)
                def _():
                    gwait(b, ci)
                    scale(b, ci)
                    sissue(b, ci)

                @pl.when(ci >= 1)
                def _():
                    swait(p, ci - 1)

                @pl.when(ci + 2 < n_chunks)
                def _():
                    gissue(p, ci + 2)
            return carry
        lax.fori_loop(0, n_iters, pipe, 0)
        plsc.subcore_barrier()

        def ocp(it, carry):
            idx = it * _NS + s
            @pl.when(idx < n_row_chunks)
            def _():
                r0 = pl.multiple_of(idx * k, 8)
                pltpu.sync_copy(acc_sh.at[pl.ds(r0, k)],
                                out_hbm.at[c, pl.ds(r0, k)])
            return carry
        lax.fori_loop(0, row_iters, ocp, 0)

    return spmm


def _spmm(x, src3, dst3, w3, n, e, d, k):
    return _make_spmm(n, e, d, k)(x, src3, dst3, w3)


_BR = 1000


def _proj(x, w):
    n, f = x.shape
    dh = w.shape[1]

    def body(x_ref, w_ref, o_ref):
        o_ref[...] = jnp.dot(x_ref[...], w_ref[...],
                             preferred_element_type=jnp.float32)

    return pl.pallas_call(
        body,
        grid=(n // _BR,),
        in_specs=[
            pl.BlockSpec((_BR, f), lambda i: (i, 0)),
            pl.BlockSpec((f, dh), lambda i: (0, 0)),
        ],
        out_specs=pl.BlockSpec((_BR, dh), lambda i: (i, 0)),
        out_shape=jax.ShapeDtypeStruct((n, dh), jnp.float32),
    )(x, w)


def _fused_trans(g, feat, gt, ft, b, w):
    _, n, dg = g.shape
    f = feat.shape[1]
    dh = gt.shape[1]
    dn = dh if w is None else w.shape[1]

    def body(*refs):
        if w is None:
            g_ref, f_ref, gt_ref, ft_ref, b_ref, o_ref = refs
        else:
            g_ref, f_ref, gt_ref, ft_ref, b_ref, w_ref, o_ref = refs
        gs = g_ref[0] + g_ref[1]
        h = jnp.dot(gs, gt_ref[...], preferred_element_type=jnp.float32)
        h += jnp.dot(f_ref[...], ft_ref[...], preferred_element_type=jnp.float32)
        h = jnp.maximum(h + b_ref[...], 0.0)
        if w is None:
            o_ref[...] = h
        else:
            o_ref[...] = jnp.dot(h, w_ref[...], preferred_element_type=jnp.float32)

    in_specs = [
        pl.BlockSpec((2, _BR, dg), lambda i: (0, i, 0)),
        pl.BlockSpec((_BR, f), lambda i: (i, 0)),
        pl.BlockSpec((dg, dh), lambda i: (0, 0)),
        pl.BlockSpec((f, dh), lambda i: (0, 0)),
        pl.BlockSpec((1, dh), lambda i: (0, 0)),
    ]
    args = [g, feat, gt, ft, b]
    if w is not None:
        in_specs.append(pl.BlockSpec((dh, dn), lambda i: (0, 0)))
        args.append(w)

    return pl.pallas_call(
        body,
        grid=(n // _BR,),
        in_specs=in_specs,
        out_specs=pl.BlockSpec((_BR, dn), lambda i: (i, 0)),
        out_shape=jax.ShapeDtypeStruct((n, dn), jnp.float32),
    )(*args)


def kernel(features, edge_index, edge_weight, W1, W2, W3, Tw, Tb, T1w, T1b, T2w, T2b):
    n, f = features.shape
    e = edge_weight.shape[0]
    h1 = W1.shape[1]
    h2 = W2.shape[1]
    h3 = W3.shape[1]
    k = 80
    n_chunks = e // _NW // k

    src3 = edge_index[0].reshape(_NW, n_chunks, k)
    dst3 = edge_index[1].reshape(_NW, n_chunks, k)
    w3 = edge_weight.reshape(_NW, n_chunks, k)

    TwgT, TwfT = Tw[:, :h1].T, Tw[:, h1:].T
    T1wgT, T1wfT = T1w[:, :h2].T, T1w[:, h2:].T
    T2wgT, T2wfT = T2w[:, :h3].T, T2w[:, h3:].T

    x1 = _proj(features, W1)
    g1a = _spmm(x1[:, :h1 // 2], src3, dst3, w3, n, e, h1 // 2, k)
    g1b = _spmm(x1[:, h1 // 2:], src3, dst3, w3, n, e, h1 // 2, k)
    g1 = jnp.concatenate([g1a, g1b], axis=2)
    x2 = _fused_trans(g1, features, TwgT, TwfT, Tb.reshape(1, h1), W2)
    g2 = _spmm(x2, src3, dst3, w3, n, e, h2, k)
    x3 = _fused_trans(g2, features, T1wgT, T1wfT, T1b.reshape(1, h2), W3)
    g3 = _spmm(x3, src3, dst3, w3, n, e, h3, k)
    return _fused_trans(g3, features, T2wgT, T2wfT, T2b.reshape(1, h3), None)


# speedup vs baseline: 7.4739x; 1.1890x over previous
import functools

import jax
import jax.numpy as jnp
from jax import lax
from jax.experimental import pallas as pl
from jax.experimental.pallas import tpu as pltpu
from jax.experimental.pallas import tpu_sc as plsc

_NC = 2
_NS = 16
_NW = _NC * _NS
_L = 16


@functools.cache
def _make_spmm(n, e, d, k):
    e_per_tile = e // _NW
    n_chunks = e_per_tile // k
    assert n_chunks * k == e_per_tile
    n_row_chunks = n // k
    assert n_row_chunks * k == n and k % 8 == 0
    row_iters = -(-n_row_chunks // _NS)

    mesh = plsc.VectorSubcoreMesh(core_axis_name="c", subcore_axis_name="s")

    @functools.partial(
        pl.kernel,
        out_type=jax.ShapeDtypeStruct((_NC, n, d), jnp.float32),
        mesh=mesh,
        scratch_types=[
            pltpu.VMEM((n_chunks, k), jnp.int32),
            pltpu.VMEM((n_chunks, k), jnp.int32),
            pltpu.VMEM((n_chunks, k), jnp.float32),
            pltpu.VMEM((k, d), jnp.bfloat16),
            pltpu.VMEM((k, d), jnp.bfloat16),
            pltpu.VMEM((k, d), jnp.bfloat16),
            pltpu.VMEM((k, d), jnp.float32),
            pltpu.VMEM((k, d), jnp.float32),
            pltpu.VMEM((k, d), jnp.float32),
            pltpu.SemaphoreType.DMA,
            pltpu.SemaphoreType.DMA,
            pltpu.SemaphoreType.DMA,
            pltpu.SemaphoreType.DMA,
            pltpu.SemaphoreType.DMA,
            pltpu.SemaphoreType.DMA,
            pltpu.VMEM_SHARED((n, d), jnp.float32),
        ],
        compiler_params=pltpu.CompilerParams(use_tc_tiling_on_sc=False,
                                             needs_layout_passes=False),
    )
    def spmm(x_hbm, src_hbm, dst_hbm, w_hbm, out_hbm,
             src_v, dst_v, w_v, braw0, braw1, braw2, rows0, rows1, rows2,
             gs0, gs1, gs2, ss0, ss1, ss2, acc_sh):
        braw = [braw0, braw1, braw2]
        rows = [rows0, rows1, rows2]
        gsem = [gs0, gs1, gs2]
        ssem = [ss0, ss1, ss2]
        rows_v = rows0
        c = lax.axis_index("c")
        s = lax.axis_index("s")
        wid = c * _NS + s

        pltpu.sync_copy(src_hbm.at[wid], src_v)
        pltpu.sync_copy(dst_hbm.at[wid], dst_v)
        pltpu.sync_copy(w_hbm.at[wid], w_v)

        def zrow(i, carry):
            for j in range(d // _L):
                rows_v[i, pl.ds(j * _L, _L)] = jnp.zeros((_L,), jnp.float32)
            return carry
        lax.fori_loop(0, k, zrow, 0)

        def zcp(it, carry):
            idx = it * _NS + s
            @pl.when(idx < n_row_chunks)
            def _():
                r0 = pl.multiple_of(idx * k, 8)
                pltpu.sync_copy(rows_v, acc_sh.at[pl.ds(r0, k)])
            return carry
        lax.fori_loop(0, row_iters, zcp, 0)
        plsc.subcore_barrier()

        def scale(b, ci):
            def body(gi, carry2):
                wvec = w_v[ci, pl.ds(gi * _L, _L)]
                for ii in range(_L):
                    w = wvec[ii]
                    row = gi * _L + ii
                    for j in range(d // (2 * _L)):
                        v = braw[b][row, pl.ds(j * 2 * _L, 2 * _L)]
                        lo, hi = plsc.unpack(v, format=plsc.PackFormat.INTERLEAVED)
                        rows[b][row, pl.ds(j * 2 * _L, _L)] = lo * w
                        rows[b][row, pl.ds(j * 2 * _L + _L, _L)] = hi * w
                return carry2
            lax.fori_loop(0, k // _L, body, 0)

        def gissue(b, ci):
            pltpu.async_copy(x_hbm.at[src_v.at[ci]], braw[b], gsem[b])

        def gwait(b, ci):
            pltpu.make_async_copy(x_hbm.at[src_v.at[ci]], braw[b], gsem[b]).wait()

        def sissue(b, ci):
            pltpu.async_copy(rows[b], acc_sh.at[dst_v.at[ci]], ssem[b], add=True)

        def swait(b, ci):
            pltpu.make_async_copy(rows[b], acc_sh.at[dst_v.at[ci]], ssem[b]).wait()

        gissue(0, 0)
        gissue(1, 1)

        n_iters = -(-(n_chunks + 1) // 3)

        def pipe(g, carry):
            for b in range(3):
                ci = g * 3 + b
                p = (b + 2) % 3

                @pl.when(ci < n_chunks)
                def _():
                    gwait(b, ci)
                    scale(b, ci)
                    sissue(b, ci)

                @pl.when(ci >= 1)
                def _():
                    swait(p, ci - 1)

                @pl.when(ci + 2 < n_chunks)
                def _():
                    gissue(p, ci + 2)
            return carry
        lax.fori_loop(0, n_iters, pipe, 0)
        plsc.subcore_barrier()

        def ocp(it, carry):
            idx = it * _NS + s
            @pl.when(idx < n_row_chunks)
            def _():
                r0 = pl.multiple_of(idx * k, 8)
                pltpu.sync_copy(acc_sh.at[pl.ds(r0, k)],
                                out_hbm.at[c, pl.ds(r0, k)])
            return carry
        lax.fori_loop(0, row_iters, ocp, 0)

    return spmm


def _spmm(x, src3, dst3, w3, n, e, d, k):
    return _make_spmm(n, e, d, k)(x, src3, dst3, w3)


_BR = 1000


def _proj(x, w):
    n, f = x.shape
    dh = w.shape[1]

    def body(x_ref, w_ref, o_ref):
        o_ref[...] = jnp.dot(x_ref[...], w_ref[...],
                             preferred_element_type=jnp.float32
                             ).astype(jnp.bfloat16)

    return pl.pallas_call(
        body,
        grid=(n // _BR,),
        in_specs=[
            pl.BlockSpec((_BR, f), lambda i: (i, 0)),
            pl.BlockSpec((f, dh), lambda i: (0, 0)),
        ],
        out_specs=pl.BlockSpec((_BR, dh), lambda i: (i, 0)),
        out_shape=jax.ShapeDtypeStruct((n, dh), jnp.bfloat16),
    )(x, w)


def _fused_trans(g, feat, gt, ft, b, w):
    _, n, dg = g.shape
    f = feat.shape[1]
    dh = gt.shape[1]
    dn = dh if w is None else w.shape[1]

    def body(*refs):
        if w is None:
            g_ref, f_ref, gt_ref, ft_ref, b_ref, o_ref = refs
        else:
            g_ref, f_ref, gt_ref, ft_ref, b_ref, w_ref, o_ref = refs
        gs = g_ref[0] + g_ref[1]
        h = jnp.dot(gs, gt_ref[...], preferred_element_type=jnp.float32)
        h += jnp.dot(f_ref[...], ft_ref[...], preferred_element_type=jnp.float32)
        h = jnp.maximum(h + b_ref[...], 0.0)
        if w is None:
            o_ref[...] = h
        else:
            o_ref[...] = jnp.dot(h, w_ref[...], preferred_element_type=jnp.float32
                                 ).astype(jnp.bfloat16)

    in_specs = [
        pl.BlockSpec((2, _BR, dg), lambda i: (0, i, 0)),
        pl.BlockSpec((_BR, f), lambda i: (i, 0)),
        pl.BlockSpec((dg, dh), lambda i: (0, 0)),
        pl.BlockSpec((f, dh), lambda i: (0, 0)),
        pl.BlockSpec((1, dh), lambda i: (0, 0)),
    ]
    args = [g, feat, gt, ft, b]
    if w is not None:
        in_specs.append(pl.BlockSpec((dh, dn), lambda i: (0, 0)))
        args.append(w)

    out_dtype = jnp.float32 if w is None else jnp.bfloat16
    return pl.pallas_call(
        body,
        grid=(n // _BR,),
        in_specs=in_specs,
        out_specs=pl.BlockSpec((_BR, dn), lambda i: (i, 0)),
        out_shape=jax.ShapeDtypeStruct((n, dn), out_dtype),
    )(*args)


def _iperm(d):
    import numpy as np
    p = np.empty((d,), dtype=np.int32)
    for j in range(d // 32):
        for t in range(16):
            p[j * 32 + 2 * t] = j * 32 + t
            p[j * 32 + 2 * t + 1] = j * 32 + 16 + t
    return p


def kernel(features, edge_index, edge_weight, W1, W2, W3, Tw, Tb, T1w, T1b, T2w, T2b):
    n, f = features.shape
    e = edge_weight.shape[0]
    h1 = W1.shape[1]
    h2 = W2.shape[1]
    h3 = W3.shape[1]
    k = 80
    n_chunks = e // _NW // k

    src3 = edge_index[0].reshape(_NW, n_chunks, k)
    dst3 = edge_index[1].reshape(_NW, n_chunks, k)
    w3 = edge_weight.reshape(_NW, n_chunks, k)

    TwgT, TwfT = Tw[:, :h1].T, Tw[:, h1:].T
    T1wgT, T1wfT = T1w[:, :h2].T, T1w[:, h2:].T
    T2wgT, T2wfT = T2w[:, :h3].T, T2w[:, h3:].T

    p64 = _iperm(h1 // 2)
    W1 = W1[:, jnp.concatenate([jnp.asarray(p64), jnp.asarray(p64) + h1 // 2])]
    W2 = W2[:, jnp.asarray(_iperm(h2))]
    W3 = W3[:, jnp.asarray(_iperm(h3))]

    x1 = _proj(features, W1)
    g1a = _spmm(x1[:, :h1 // 2], src3, dst3, w3, n, e, h1 // 2, k)
    g1b = _spmm(x1[:, h1 // 2:], src3, dst3, w3, n, e, h1 // 2, k)
    g1 = jnp.concatenate([g1a, g1b], axis=2)
    x2 = _fused_trans(g1, features, TwgT, TwfT, Tb.reshape(1, h1), W2)
    g2 = _spmm(x2, src3, dst3, w3, n, e, h2, k)
    x3 = _fused_trans(g2, features, T1wgT, T1wfT, T1b.reshape(1, h2), W3)
    g3 = _spmm(x3, src3, dst3, w3, n, e, h3, k)
    return _fused_trans(g3, features, T2wgT, T2wfT, T2b.reshape(1, h3), None)


# speedup vs baseline: 7.8512x; 1.0505x over previous
import functools

import jax
import jax.numpy as jnp
from jax import lax
from jax.experimental import pallas as pl
from jax.experimental.pallas import tpu as pltpu
from jax.experimental.pallas import tpu_sc as plsc

_NC = 2
_NS = 16
_NW = _NC * _NS
_L = 16


@functools.cache
def _make_spmm(n, e, d, k, bf16=True):
    e_per_tile = e // _NW
    n_chunks = e_per_tile // k
    assert n_chunks * k == e_per_tile
    n_row_chunks = n // k
    assert n_row_chunks * k == n and k % 8 == 0
    row_iters = -(-n_row_chunks // _NS)

    mesh = plsc.VectorSubcoreMesh(core_axis_name="c", subcore_axis_name="s")

    @functools.partial(
        pl.kernel,
        out_type=jax.ShapeDtypeStruct((_NC, n, d), jnp.float32),
        mesh=mesh,
        scratch_types=[
            pltpu.VMEM((n_chunks, k), jnp.int32),
            pltpu.VMEM((n_chunks, k), jnp.int32),
            pltpu.VMEM((n_chunks, k), jnp.float32),
            pltpu.VMEM((k, d), jnp.bfloat16 if bf16 else jnp.float32),
            pltpu.VMEM((k, d), jnp.bfloat16 if bf16 else jnp.float32),
            pltpu.VMEM((k, d), jnp.bfloat16 if bf16 else jnp.float32),
            pltpu.VMEM((k, d), jnp.float32),
            pltpu.VMEM((k, d), jnp.float32),
            pltpu.VMEM((k, d), jnp.float32),
            pltpu.SemaphoreType.DMA,
            pltpu.SemaphoreType.DMA,
            pltpu.SemaphoreType.DMA,
            pltpu.SemaphoreType.DMA,
            pltpu.SemaphoreType.DMA,
            pltpu.SemaphoreType.DMA,
            pltpu.VMEM_SHARED((n, d), jnp.float32),
        ],
        compiler_params=pltpu.CompilerParams(use_tc_tiling_on_sc=False,
                                             needs_layout_passes=False),
    )
    def spmm(x_hbm, src_hbm, dst_hbm, w_hbm, out_hbm,
             src_v, dst_v, w_v, braw0, braw1, braw2, rows0, rows1, rows2,
             gs0, gs1, gs2, ss0, ss1, ss2, acc_sh):
        braw = [braw0, braw1, braw2]
        rows = [rows0, rows1, rows2]
        gsem = [gs0, gs1, gs2]
        ssem = [ss0, ss1, ss2]
        rows_v = rows0
        c = lax.axis_index("c")
        s = lax.axis_index("s")
        wid = c * _NS + s

        pltpu.sync_copy(src_hbm.at[wid], src_v)
        pltpu.sync_copy(dst_hbm.at[wid], dst_v)
        pltpu.sync_copy(w_hbm.at[wid], w_v)

        def zrow(i, carry):
            for j in range(d // _L):
                rows_v[i, pl.ds(j * _L, _L)] = jnp.zeros((_L,), jnp.float32)
            return carry
        lax.fori_loop(0, k, zrow, 0)

        def zcp(it, carry):
            idx = it * _NS + s
            @pl.when(idx < n_row_chunks)
            def _():
                r0 = pl.multiple_of(idx * k, 8)
                pltpu.sync_copy(rows_v, acc_sh.at[pl.ds(r0, k)])
            return carry
        lax.fori_loop(0, row_iters, zcp, 0)
        plsc.subcore_barrier()

        def scale(b, ci):
            def body(gi, carry2):
                wvec = w_v[ci, pl.ds(gi * _L, _L)]
                for ii in range(_L):
                    w = wvec[ii]
                    row = gi * _L + ii
                    if bf16:
                        for j in range(d // (2 * _L)):
                            v = braw[b][row, pl.ds(j * 2 * _L, 2 * _L)]
                            lo, hi = plsc.unpack(
                                v, format=plsc.PackFormat.INTERLEAVED)
                            rows[b][row, pl.ds(j * 2 * _L, _L)] = lo * w
                            rows[b][row, pl.ds(j * 2 * _L + _L, _L)] = hi * w
                    else:
                        for j in range(d // _L):
                            sl = pl.ds(j * _L, _L)
                            rows[b][row, sl] = braw[b][row, sl] * w
                return carry2
            lax.fori_loop(0, k // _L, body, 0)

        def gissue(b, ci):
            pltpu.async_copy(x_hbm.at[src_v.at[ci]], braw[b], gsem[b])

        def gwait(b, ci):
            pltpu.make_async_copy(x_hbm.at[src_v.at[ci]], braw[b], gsem[b]).wait()

        def sissue(b, ci):
            pltpu.async_copy(rows[b], acc_sh.at[dst_v.at[ci]], ssem[b], add=True)

        def swait(b, ci):
            pltpu.make_async_copy(rows[b], acc_sh.at[dst_v.at[ci]], ssem[b]).wait()

        gissue(0, 0)
        gissue(1, 1)

        n_iters = -(-(n_chunks + 1) // 3)

        def pipe(g, carry):
            for b in range(3):
                ci = g * 3 + b
                p = (b + 2) % 3

                @pl.when(ci < n_chunks)
                def _():
                    gwait(b, ci)
                    scale(b, ci)
                    sissue(b, ci)

                @pl.when(ci >= 1)
                def _():
                    swait(p, ci - 1)

                @pl.when(ci + 2 < n_chunks)
                def _():
                    gissue(p, ci + 2)
            return carry
        lax.fori_loop(0, n_iters, pipe, 0)
        plsc.subcore_barrier()

        def ocp(it, carry):
            idx = it * _NS + s
            @pl.when(idx < n_row_chunks)
            def _():
                r0 = pl.multiple_of(idx * k, 8)
                pltpu.sync_copy(acc_sh.at[pl.ds(r0, k)],
                                out_hbm.at[c, pl.ds(r0, k)])
            return carry
        lax.fori_loop(0, row_iters, ocp, 0)

    return spmm


def _spmm(x, src3, dst3, w3, n, e, d, k):
    return _make_spmm(n, e, d, k, x.dtype == jnp.bfloat16)(x, src3, dst3, w3)


_BR = 1000


def _proj(x, w):
    n, f = x.shape
    dh = w.shape[1]

    def body(x_ref, w_ref, o_ref):
        o_ref[...] = jnp.dot(x_ref[...], w_ref[...],
                             preferred_element_type=jnp.float32
                             ).astype(jnp.bfloat16)

    return pl.pallas_call(
        body,
        grid=(n // _BR,),
        in_specs=[
            pl.BlockSpec((_BR, f), lambda i: (i, 0)),
            pl.BlockSpec((f, dh), lambda i: (0, 0)),
        ],
        out_specs=pl.BlockSpec((_BR, dh), lambda i: (i, 0)),
        out_shape=jax.ShapeDtypeStruct((n, dh), jnp.bfloat16),
    )(x, w)


def _fused_trans(gs, feat, gts, ft, b, w, out_dtype):
    ng = len(gs)
    n = gs[0].shape[1]
    f = feat.shape[1]
    dh = gts[0].shape[1]
    dn = dh if w is None else w.shape[1]

    def body(*refs):
        g_refs = refs[:ng]
        gt_refs = refs[ng + 2:2 * ng + 2]
        f_ref, ft_ref = refs[ng], refs[ng + 1]
        b_ref = refs[2 * ng + 2]
        o_ref = refs[-1]
        h = jnp.dot(f_ref[...], ft_ref[...], preferred_element_type=jnp.float32)
        for g_ref, gt_ref in zip(g_refs, gt_refs):
            h += jnp.dot(g_ref[0] + g_ref[1], gt_ref[...],
                         preferred_element_type=jnp.float32)
        h = jnp.maximum(h + b_ref[...], 0.0)
        if w is None:
            o_ref[...] = h.astype(out_dtype)
        else:
            w_ref = refs[2 * ng + 3]
            o_ref[...] = jnp.dot(h, w_ref[...], preferred_element_type=jnp.float32
                                 ).astype(out_dtype)

    in_specs = (
        [pl.BlockSpec((2, _BR, g.shape[2]), lambda i: (0, i, 0)) for g in gs]
        + [pl.BlockSpec((_BR, f), lambda i: (i, 0)),
           pl.BlockSpec((f, dh), lambda i: (0, 0))]
        + [pl.BlockSpec((gt.shape[0], dh), lambda i: (0, 0)) for gt in gts]
        + [pl.BlockSpec((1, dh), lambda i: (0, 0))]
    )
    args = list(gs) + [feat, ft] + list(gts) + [b]
    if w is not None:
        in_specs.append(pl.BlockSpec((dh, dn), lambda i: (0, 0)))
        args.append(w)

    return pl.pallas_call(
        body,
        grid=(n // _BR,),
        in_specs=in_specs,
        out_specs=pl.BlockSpec((_BR, dn), lambda i: (i, 0)),
        out_shape=jax.ShapeDtypeStruct((n, dn), out_dtype),
    )(*args)


def _iperm(d):
    import numpy as np
    p = np.empty((d,), dtype=np.int32)
    for j in range(d // 32):
        for t in range(16):
            p[j * 32 + 2 * t] = j * 32 + t
            p[j * 32 + 2 * t + 1] = j * 32 + 16 + t
    return p


def kernel(features, edge_index, edge_weight, W1, W2, W3, Tw, Tb, T1w, T1b, T2w, T2b):
    n, f = features.shape
    e = edge_weight.shape[0]
    h1 = W1.shape[1]
    h2 = W2.shape[1]
    h3 = W3.shape[1]
    k = 80
    n_chunks = e // _NW // k

    src3 = edge_index[0].reshape(_NW, n_chunks, k)
    dst3 = edge_index[1].reshape(_NW, n_chunks, k)
    w3 = edge_weight.reshape(_NW, n_chunks, k)

    TwgT, TwfT = Tw[:, :h1].T, Tw[:, h1:].T
    T1wgT, T1wfT = T1w[:, :h2].T, T1w[:, h2:].T
    T2wgT, T2wfT = T2w[:, :h3].T, T2w[:, h3:].T

    p64 = _iperm(h1 // 2)
    W1 = W1[:, jnp.concatenate([jnp.asarray(p64), jnp.asarray(p64) + h1 // 2])]
    W2 = W2[:, jnp.asarray(_iperm(h2))]

    x1 = _proj(features, W1)
    g1a = _spmm(x1[:, :h1 // 2], src3, dst3, w3, n, e, h1 // 2, k)
    g1b = _spmm(x1[:, h1 // 2:], src3, dst3, w3, n, e, h1 // 2, k)
    x2 = _fused_trans([g1a, g1b], features, [TwgT[:h1 // 2], TwgT[h1 // 2:]],
                      TwfT, Tb.reshape(1, h1), W2, jnp.bfloat16)
    g2 = _spmm(x2, src3, dst3, w3, n, e, h2, k)
    x3 = _fused_trans([g2], features, [T1wgT], T1wfT, T1b.reshape(1, h2), W3,
                      jnp.float32)
    g3 = _spmm(x3, src3, dst3, w3, n, e, h3, k)
    return _fused_trans([g3], features, [T2wgT], T2wfT, T2b.reshape(1, h3),
                        None, jnp.float32)


# speedup vs baseline: 10.8335x; 1.3798x over previous
import functools

import jax
import jax.numpy as jnp
from jax import lax
from jax.experimental import pallas as pl
from jax.experimental.pallas import tpu as pltpu
from jax.experimental.pallas import tpu_sc as plsc

_NC = 2
_NS = 16
_NW = _NC * _NS
_L = 16


@functools.cache
def _make_spmm(n, e, d, k, bf16=True):
    e_per_tile = e // _NW
    n_chunks = e_per_tile // k
    assert n_chunks * k == e_per_tile
    n_row_chunks = n // k
    assert n_row_chunks * k == n and k % 8 == 0
    row_iters = -(-n_row_chunks // _NS)

    mesh = plsc.VectorSubcoreMesh(core_axis_name="c", subcore_axis_name="s")

    @functools.partial(
        pl.kernel,
        out_type=jax.ShapeDtypeStruct((_NC, n, d), jnp.bfloat16),
        mesh=mesh,
        scratch_types=[
            pltpu.VMEM((n_chunks, k), jnp.int32),
            pltpu.VMEM((n_chunks, k), jnp.int32),
            pltpu.VMEM((n_chunks, k), jnp.float32),
            pltpu.VMEM((k, d), jnp.bfloat16 if bf16 else jnp.float32),
            pltpu.VMEM((k, d), jnp.bfloat16 if bf16 else jnp.float32),
            pltpu.VMEM((k, d), jnp.bfloat16 if bf16 else jnp.float32),
            pltpu.VMEM((k, d), jnp.bfloat16),
            pltpu.VMEM((k, d), jnp.bfloat16),
            pltpu.VMEM((k, d), jnp.bfloat16),
            pltpu.SemaphoreType.DMA,
            pltpu.SemaphoreType.DMA,
            pltpu.SemaphoreType.DMA,
            pltpu.SemaphoreType.DMA,
            pltpu.SemaphoreType.DMA,
            pltpu.SemaphoreType.DMA,
            pltpu.VMEM_SHARED((n, d), jnp.bfloat16),
        ],
        compiler_params=pltpu.CompilerParams(use_tc_tiling_on_sc=False,
                                             needs_layout_passes=False),
    )
    def spmm(x_hbm, src_hbm, dst_hbm, w_hbm, out_hbm,
             src_v, dst_v, w_v, braw0, braw1, braw2, rows0, rows1, rows2,
             gs0, gs1, gs2, ss0, ss1, ss2, acc_sh):
        braw = [braw0, braw1, braw2]
        rows = [rows0, rows1, rows2]
        gsem = [gs0, gs1, gs2]
        ssem = [ss0, ss1, ss2]
        rows_v = rows0
        c = lax.axis_index("c")
        s = lax.axis_index("s")
        wid = c * _NS + s

        pltpu.sync_copy(src_hbm.at[wid], src_v)
        pltpu.sync_copy(dst_hbm.at[wid], dst_v)
        pltpu.sync_copy(w_hbm.at[wid], w_v)

        def zrow(i, carry):
            for j in range(d // (2 * _L)):
                rows_v[i, pl.ds(j * 2 * _L, 2 * _L)] = jnp.zeros(
                    (2 * _L,), jnp.bfloat16)
            return carry
        lax.fori_loop(0, k, zrow, 0)

        def zcp(it, carry):
            idx = it * _NS + s
            @pl.when(idx < n_row_chunks)
            def _():
                r0 = pl.multiple_of(idx * k, 8)
                pltpu.sync_copy(rows_v, acc_sh.at[pl.ds(r0, k)])
            return carry
        lax.fori_loop(0, row_iters, zcp, 0)
        plsc.subcore_barrier()

        def scale(b, ci):
            def body(gi, carry2):
                wvec = w_v[ci, pl.ds(gi * _L, _L)]
                for ii in range(_L):
                    w = wvec[ii]
                    row = gi * _L + ii
                    for j in range(d // (2 * _L)):
                        sl = pl.ds(j * 2 * _L, 2 * _L)
                        if bf16:
                            lo, hi = plsc.unpack(
                                braw[b][row, sl],
                                format=plsc.PackFormat.INTERLEAVED)
                        else:
                            lo = braw[b][row, pl.ds(j * 2 * _L, _L)]
                            hi = braw[b][row, pl.ds(j * 2 * _L + _L, _L)]
                        rows[b][row, sl] = plsc.pack(
                            lo * w, hi * w, format=plsc.PackFormat.INTERLEAVED)
                return carry2
            lax.fori_loop(0, k // _L, body, 0)

        def gissue(b, ci):
            pltpu.async_copy(x_hbm.at[src_v.at[ci]], braw[b], gsem[b])

        def gwait(b, ci):
            pltpu.make_async_copy(x_hbm.at[src_v.at[ci]], braw[b], gsem[b]).wait()

        def sissue(b, ci):
            pltpu.async_copy(rows[b], acc_sh.at[dst_v.at[ci]], ssem[b], add=True)

        def swait(b, ci):
            pltpu.make_async_copy(rows[b], acc_sh.at[dst_v.at[ci]], ssem[b]).wait()

        gissue(0, 0)
        gissue(1, 1)

        n_iters = -(-(n_chunks + 1) // 3)

        def pipe(g, carry):
            for b in range(3):
                ci = g * 3 + b
                p = (b + 2) % 3

                @pl.when(ci < n_chunks)
                def _():
                    gwait(b, ci)
                    scale(b, ci)
                    sissue(b, ci)

                @pl.when(ci >= 1)
                def _():
                    swait(p, ci - 1)

                @pl.when(ci + 2 < n_chunks)
                def _():
                    gissue(p, ci + 2)
            return carry
        lax.fori_loop(0, n_iters, pipe, 0)
        plsc.subcore_barrier()

        def ocp(it, carry):
            idx = it * _NS + s
            @pl.when(idx < n_row_chunks)
            def _():
                r0 = pl.multiple_of(idx * k, 8)
                pltpu.sync_copy(acc_sh.at[pl.ds(r0, k)],
                                out_hbm.at[c, pl.ds(r0, k)])
            return carry
        lax.fori_loop(0, row_iters, ocp, 0)

    return spmm


def _spmm(x, src3, dst3, w3, n, e, d, k):
    return _make_spmm(n, e, d, k, x.dtype == jnp.bfloat16)(x, src3, dst3, w3)


_BR = 1000


def _proj(x, w):
    n, f = x.shape
    dh = w.shape[1]

    def body(x_ref, w_ref, o_ref):
        o_ref[...] = jnp.dot(x_ref[...], w_ref[...],
                             preferred_element_type=jnp.float32
                             ).astype(jnp.bfloat16)

    return pl.pallas_call(
        body,
        grid=(n // _BR,),
        in_specs=[
            pl.BlockSpec((_BR, f), lambda i: (i, 0)),
            pl.BlockSpec((f, dh), lambda i: (0, 0)),
        ],
        out_specs=pl.BlockSpec((_BR, dh), lambda i: (i, 0)),
        out_shape=jax.ShapeDtypeStruct((n, dh), jnp.bfloat16),
    )(x, w)


def _fused_trans(gs, feat, gts, ft, b, w, out_dtype):
    ng = len(gs)
    n = gs[0].shape[1]
    f = feat.shape[1]
    dh = gts[0].shape[1]
    dn = dh if w is None else w.shape[1]

    def body(*refs):
        g_refs = refs[:ng]
        gt_refs = refs[ng + 2:2 * ng + 2]
        f_ref, ft_ref = refs[ng], refs[ng + 1]
        b_ref = refs[2 * ng + 2]
        o_ref = refs[-1]
        h = jnp.dot(f_ref[...], ft_ref[...], preferred_element_type=jnp.float32)
        for g_ref, gt_ref in zip(g_refs, gt_refs):
            gsum = g_ref[0].astype(jnp.float32) + g_ref[1].astype(jnp.float32)
            h += jnp.dot(gsum, gt_ref[...], preferred_element_type=jnp.float32)
        h = jnp.maximum(h + b_ref[...], 0.0)
        if w is None:
            o_ref[...] = h.astype(out_dtype)
        else:
            w_ref = refs[2 * ng + 3]
            o_ref[...] = jnp.dot(h, w_ref[...], preferred_element_type=jnp.float32
                                 ).astype(out_dtype)

    in_specs = (
        [pl.BlockSpec((2, _BR, g.shape[2]), lambda i: (0, i, 0)) for g in gs]
        + [pl.BlockSpec((_BR, f), lambda i: (i, 0)),
           pl.BlockSpec((f, dh), lambda i: (0, 0))]
        + [pl.BlockSpec((gt.shape[0], dh), lambda i: (0, 0)) for gt in gts]
        + [pl.BlockSpec((1, dh), lambda i: (0, 0))]
    )
    args = list(gs) + [feat, ft] + list(gts) + [b]
    if w is not None:
        in_specs.append(pl.BlockSpec((dh, dn), lambda i: (0, 0)))
        args.append(w)

    return pl.pallas_call(
        body,
        grid=(n // _BR,),
        in_specs=in_specs,
        out_specs=pl.BlockSpec((_BR, dn), lambda i: (i, 0)),
        out_shape=jax.ShapeDtypeStruct((n, dn), out_dtype),
    )(*args)


def _iperm(d):
    import numpy as np
    p = np.empty((d,), dtype=np.int32)
    for j in range(d // 32):
        for t in range(16):
            p[j * 32 + 2 * t] = j * 32 + t
            p[j * 32 + 2 * t + 1] = j * 32 + 16 + t
    return p


def kernel(features, edge_index, edge_weight, W1, W2, W3, Tw, Tb, T1w, T1b, T2w, T2b):
    n, f = features.shape
    e = edge_weight.shape[0]
    h1 = W1.shape[1]
    h2 = W2.shape[1]
    h3 = W3.shape[1]
    k = 80
    n_chunks = e // _NW // k

    src3 = edge_index[0].reshape(_NW, n_chunks, k)
    dst3 = edge_index[1].reshape(_NW, n_chunks, k)
    w3 = edge_weight.reshape(_NW, n_chunks, k)

    TwgT, TwfT = Tw[:, :h1].T, Tw[:, h1:].T
    T1wgT, T1wfT = T1w[:, :h2].T, T1w[:, h2:].T
    T2wgT, T2wfT = T2w[:, :h3].T, T2w[:, h3:].T

    p64 = _iperm(h1 // 2)
    W1 = W1[:, jnp.concatenate([jnp.asarray(p64), jnp.asarray(p64) + h1 // 2])]
    W2 = W2[:, jnp.asarray(_iperm(h2))]
    TwgT = TwgT[jnp.concatenate([jnp.asarray(p64), jnp.asarray(p64) + h1 // 2])]
    T1wgT = T1wgT[jnp.asarray(_iperm(h2))]
    T2wgT = T2wgT[jnp.asarray(_iperm(h3))]

    x1 = _proj(features, W1)
    g1a = _spmm(x1[:, :h1 // 2], src3, dst3, w3, n, e, h1 // 2, k)
    g1b = _spmm(x1[:, h1 // 2:], src3, dst3, w3, n, e, h1 // 2, k)
    x2 = _fused_trans([g1a, g1b], features, [TwgT[:h1 // 2], TwgT[h1 // 2:]],
                      TwfT, Tb.reshape(1, h1), W2, jnp.bfloat16)
    g2 = _spmm(x2, src3, dst3, w3, n, e, h2, k)
    x3 = _fused_trans([g2], features, [T1wgT], T1wfT, T1b.reshape(1, h2), W3,
                      jnp.float32)
    g3 = _spmm(x3, src3, dst3, w3, n, e, h3, k)
    return _fused_trans([g3], features, [T2wgT], T2wfT, T2b.reshape(1, h3),
                        None, jnp.float32)
